# baseline (speedup 1.0000x reference)
import functools

import jax
import jax.numpy as jnp
from jax import lax
from jax.experimental import pallas as pl
from jax.experimental.pallas import tpu as pltpu, tpu_sc as plsc

_ROWS = 4096
_LANES = 128
_NSCORE = 10


def _tc_x_kernel(x_ref, es_w_ref, es_b_ref, bl_w_ref, bl_b_ref,
                 base_ref, score_ref):
    sub = _ROWS // _LANES
    xb = x_ref[...]
    dn_t = (((0,), (1,)), ((), ()))
    scores = jax.lax.dot_general(es_w_ref[...], xb, dn_t,
                                 preferred_element_type=jnp.float32)
    scores = scores + es_b_ref[...]
    baseline = jax.lax.dot_general(bl_w_ref[...], xb, dn_t,
                                   preferred_element_type=jnp.float32)
    baseline = baseline + bl_b_ref[0, 0]
    base_ref[...] = baseline.reshape(sub, _LANES)
    for j in range(_NSCORE):
        score_ref[j] = scores[j:j + 1, :].reshape(sub, _LANES)


def _tc_temp_kernel(t_ref, temp_ref,
                    te_w1_ref, te_b1_ref, te_w2_ref, te_b2_ref,
                    alpha_w_ref, alpha_b_ref, beta_w_ref, beta_b_ref,
                    gw1_t_ref, gw1_e_ref, gate_b1_ref, gate_w2_ref,
                    gate_b2_ref, k_vec_ref,
                    tempc_ref, seas_ref, gate_ref):
    tn = t_ref[...] * (1.0 / 168.0)
    tp = temp_ref[...]

    h = [jnp.maximum(tp * te_w1_ref[0, j] + te_b1_ref[0, j], 0.0)
         for j in range(16)]
    te = [te_b2_ref[0, k] + sum(h[j] * te_w2_ref[j, k] for j in range(16))
          for k in range(10)]

    seasonal = jnp.zeros_like(tn)
    for c in range(4):
        alpha_c = alpha_b_ref[0, c] + sum(te[k] * alpha_w_ref[k, c]
                                          for k in range(10))
        beta_c = beta_b_ref[0, c] + sum(te[k] * beta_w_ref[k, c]
                                        for k in range(10))
        harm_c = (2.0 * jnp.pi) * k_vec_ref[0, c] * tn
        seasonal = seasonal + alpha_c * jnp.sin(harm_c) + beta_c * jnp.cos(harm_c)

    gate = jnp.zeros_like(tn)
    for j in range(16):
        gh_j = jnp.maximum(tn * gw1_t_ref[0, j]
                           + sum(te[k] * gw1_e_ref[k, j] for k in range(10))
                           + gate_b1_ref[0, j], 0.0)
        gate = gate + gh_j * gate_w2_ref[j, 0]
    gate = jax.nn.sigmoid(gate + gate_b2_ref[0, 0])

    tempc_ref[...] = gate * seasonal
    seas_ref[...] = seasonal
    gate_ref[...] = gate


def _sc_event_kernel(rows, s_hbm, eew_hbm, eeb_hbm, event_hbm,
                     s_v, eew_v, eeb_v, event_v, sem):
    nc = plsc.get_sparse_core_info().num_cores
    wid = lax.axis_index("s") * nc + lax.axis_index("c")
    base = wid * rows

    cps = [pltpu.async_copy(s_hbm.at[:, pl.ds(base, rows), :], s_v, sem),
           pltpu.async_copy(eew_hbm, eew_v, sem),
           pltpu.async_copy(eeb_hbm, eeb_v, sem)]
    for cp in cps:
        cp.wait()

    neg_inf = jnp.full((16,), -jnp.inf, jnp.float32)
    zeros16 = jnp.zeros((16,), jnp.float32)
    ones16 = jnp.ones((16,), jnp.float32)

    w = [eew_v[j, :] for j in range(_NSCORE)]
    for r in range(rows):
        for l in range(_LANES // 16):
            off = l * 16
            s = [s_v[j, r, pl.ds(off, 16)] for j in range(_NSCORE)]
            m1 = s[0]
            for j in range(1, _NSCORE):
                m1 = jnp.maximum(m1, s[j])
            found = zeros16
            w1 = w[0]
            s2 = []
            for j in range(_NSCORE):
                eq = jnp.where(s[j] == m1, ones16, zeros16)
                cond = eq * (1.0 - found)
                w1 = w1 + cond * (w[j] - w1)
                s2.append(jnp.where(cond > 0.5, neg_inf, s[j]))
                found = jnp.maximum(found, eq)
            m2 = s2[0]
            for j in range(1, _NSCORE):
                m2 = jnp.maximum(m2, s2[j])
            found2 = zeros16
            w2 = w[0]
            for j in range(_NSCORE):
                eq = jnp.where(s2[j] == m2, ones16, zeros16)
                cond = eq * (1.0 - found2)
                w2 = w2 + cond * (w[j] - w2)
                found2 = jnp.maximum(found2, eq)
            event_v[r, pl.ds(off, 16)] = m1 * w1 + m2 * w2 + eeb_v[...]

    pltpu.async_copy(event_v, event_hbm.at[pl.ds(base, rows), :], sem).wait()


def _sc_combine_kernel(rows, base_hbm, tempc_hbm, event_hbm, out_hbm,
                       base_v, tempc_v, event_v, out_v, sem):
    nc = plsc.get_sparse_core_info().num_cores
    wid = lax.axis_index("s") * nc + lax.axis_index("c")
    base = wid * rows

    cps = [pltpu.async_copy(base_hbm.at[pl.ds(base, rows), :], base_v, sem),
           pltpu.async_copy(tempc_hbm.at[pl.ds(base, rows), :], tempc_v, sem),
           pltpu.async_copy(event_hbm.at[pl.ds(base, rows), :], event_v, sem)]
    for cp in cps:
        cp.wait()

    for r in range(rows):
        for l in range(_LANES // 16):
            off = l * 16
            out_v[r, pl.ds(off, 16)] = (base_v[r, pl.ds(off, 16)]
                                        + tempc_v[r, pl.ds(off, 16)]
                                        + event_v[r, pl.ds(off, 16)])

    pltpu.async_copy(out_v, out_hbm.at[pl.ds(base, rows), :], sem).wait()


@jax.jit
def kernel(x, t, temp, te_w1, te_b1, te_w2, te_b2, alpha_w, alpha_b,
           beta_w, beta_b, gate_w1, gate_b1, gate_w2, gate_b2, k_vector,
           es_w, es_b, ee_w, ee_b, bl_w, bl_b):
    B = x.shape[0]
    R = _ROWS
    sub = R // _LANES
    BD = B // _LANES

    t2 = t.reshape(BD, _LANES)
    temp2 = temp.reshape(BD, _LANES)

    te_b1_2 = te_b1.reshape(1, -1)
    te_b2_2 = te_b2.reshape(1, -1)
    alpha_b_2 = alpha_b.reshape(1, -1)
    beta_b_2 = beta_b.reshape(1, -1)
    gw1_t = gate_w1[0:1, :]
    gw1_e = gate_w1[1:, :]
    gate_b1_2 = gate_b1.reshape(1, -1)
    gate_b2_2 = gate_b2.reshape(1, -1)
    es_b_2 = es_b.reshape(-1, 1)
    bl_b_2 = bl_b.reshape(1, -1)

    def whole(a):
        return pl.BlockSpec(a.shape, lambda i: (0, 0))

    dense_spec = pl.BlockSpec((sub, _LANES), lambda i: (i, 0))
    dense_shape = jax.ShapeDtypeStruct((BD, _LANES), jnp.float32)

    base_d, scores_d = pl.pallas_call(
        _tc_x_kernel,
        grid=(B // R,),
        in_specs=[pl.BlockSpec((R, x.shape[1]), lambda i: (i, 0)),
                  whole(es_w), whole(es_b_2), whole(bl_w), whole(bl_b_2)],
        out_specs=(dense_spec,
                   pl.BlockSpec((_NSCORE, sub, _LANES), lambda i: (0, i, 0))),
        out_shape=(dense_shape,
                   jax.ShapeDtypeStruct((_NSCORE, BD, _LANES), jnp.float32)),
    )(x, es_w, es_b_2, bl_w, bl_b_2)

    tsmall = [te_w1, te_b1_2, te_w2, te_b2_2, alpha_w, alpha_b_2, beta_w,
              beta_b_2, gw1_t, gw1_e, gate_b1_2, gate_w2, gate_b2_2,
              k_vector]
    full_spec = pl.BlockSpec((BD, _LANES), lambda i: (i, 0))
    tempc_d, seas_d, gate_d = pl.pallas_call(
        _tc_temp_kernel,
        grid=(1,),
        in_specs=[full_spec, full_spec] + [whole(a) for a in tsmall],
        out_specs=(full_spec,) * 3,
        out_shape=(dense_shape,) * 3,
    )(t2, temp2, *tsmall)

    info = plsc.get_sparse_core_info()
    nw = info.num_cores * info.num_subcores
    rows = BD // nw
    eew_b = jnp.broadcast_to(ee_w.reshape(_NSCORE, 1), (_NSCORE, 16))
    eeb_b = jnp.broadcast_to(ee_b.reshape(1), (16,))
    mesh = plsc.VectorSubcoreMesh(core_axis_name="c", subcore_axis_name="s")

    event_d = pl.kernel(
        functools.partial(_sc_event_kernel, rows),
        out_type=jax.ShapeDtypeStruct((BD, _LANES), jnp.float32),
        mesh=mesh,
        scratch_types=[
            pltpu.VMEM((_NSCORE, rows, _LANES), jnp.float32),
            pltpu.VMEM((_NSCORE, 16), jnp.float32),
            pltpu.VMEM((16,), jnp.float32),
            pltpu.VMEM((rows, _LANES), jnp.float32),
            pltpu.SemaphoreType.DMA,
        ],
    )(scores_d, eew_b, eeb_b)

    out_d = pl.kernel(
        functools.partial(_sc_combine_kernel, rows),
        out_type=jax.ShapeDtypeStruct((BD, _LANES), jnp.float32),
        mesh=mesh,
        scratch_types=[
            pltpu.VMEM((rows, _LANES), jnp.float32),
            pltpu.VMEM((rows, _LANES), jnp.float32),
            pltpu.VMEM((rows, _LANES), jnp.float32),
            pltpu.VMEM((rows, _LANES), jnp.float32),
            pltpu.SemaphoreType.DMA,
        ],
    )(base_d, tempc_d, event_d)

    return (out_d.reshape(B, 1), base_d.reshape(B, 1),
            tempc_d.reshape(B, 1), event_d.reshape(B, 1),
            seas_d.reshape(B, 1), gate_d.reshape(B, 1))
